# baseline (speedup 1.0000x reference)
import functools

import jax
import jax.numpy as jnp
from jax import lax
from jax.experimental import pallas as pl
from jax.experimental.pallas import tpu as pltpu
from jax.experimental.pallas import tpu_sc as plsc

N = 2048
D = 768
H = 3072
E = 8
TILE = 256
MAXT = 24
S = MAXT * TILE
NSC = 32


def _k1_body(gi_ref, wg_ref, d1_ref, d2_ref, g1_ref, g2_ref, te_ref):
    logits = jnp.dot(gi_ref[...], wg_ref[...],
                     preferred_element_type=jnp.float32)
    iota_e = lax.broadcasted_iota(jnp.int32, (N, E), 1)
    m1 = jnp.max(logits, axis=1, keepdims=True)
    i1 = jnp.min(jnp.where(logits == m1, iota_e, E), axis=1, keepdims=True)
    mask1 = iota_e == i1
    logits2 = jnp.where(mask1, -jnp.inf, logits)
    m2 = jnp.max(logits2, axis=1, keepdims=True)
    i2 = jnp.min(jnp.where(logits2 == m2, iota_e, E), axis=1, keepdims=True)
    mask2 = iota_e == i2
    t = jnp.exp(m2 - m1)
    g1 = 1.0 / (1.0 + t)
    g2 = 1.0 - g1
    oh1 = mask1.astype(jnp.float32)
    oh2 = mask2.astype(jnp.float32)

    def incl_cumsum_rows(a):
        sh = 1
        while sh < N:
            a = a + jnp.concatenate(
                [jnp.zeros((sh, E), jnp.float32), a[: N - sh]], axis=0)
            sh *= 2
        return a

    c1 = incl_cumsum_rows(oh1)
    c2 = incl_cumsum_rows(oh2)
    csum1 = c1 - oh1
    csum2 = c2 - oh2
    cnt1 = c1[N - 1:N, :]
    cnt = cnt1 + c2[N - 1:N, :]
    cnt_i = cnt.astype(jnp.int32)
    pc = (((cnt_i + (TILE - 1)) >> 8) << 8).astype(jnp.float32)
    pi = pc
    sh = 1
    while sh < E:
        pi = pi + jnp.concatenate(
            [jnp.zeros((1, sh), jnp.float32), pi[:, : E - sh]], axis=1)
        sh *= 2
    offpad = pi - pc
    ends = pi
    dest1 = jnp.sum(oh1 * (offpad + csum1), axis=1, keepdims=True)
    dest2 = jnp.sum(oh2 * (offpad + cnt1 + csum2), axis=1, keepdims=True)
    d1_ref[...] = dest1.astype(jnp.int32)
    d2_ref[...] = dest2.astype(jnp.int32)
    g1_ref[...] = g1
    g2_ref[...] = g2
    tstart = (lax.broadcasted_iota(jnp.int32, (MAXT, 1), 0) * TILE
              ).astype(jnp.float32)
    te_ref[...] = jnp.sum((tstart >= ends).astype(jnp.int32), axis=1,
                          keepdims=True)


def _k1(gate_inp, w_gate):
    return pl.pallas_call(
        _k1_body,
        out_shape=(
            jax.ShapeDtypeStruct((N, 1), jnp.int32),
            jax.ShapeDtypeStruct((N, 1), jnp.int32),
            jax.ShapeDtypeStruct((N, 1), jnp.float32),
            jax.ShapeDtypeStruct((N, 1), jnp.float32),
            jax.ShapeDtypeStruct((MAXT, 1), jnp.int32),
        ),
    )(gate_inp, w_gate)


@functools.cache
def _sc_mesh():
    return plsc.VectorSubcoreMesh(core_axis_name="c", subcore_axis_name="s")


def _k2a_body(d1, d2, g1, g2, st_out, sg_out, dst_v, g_v, st_v, sg_v):
    c = lax.axis_index("c")
    s = lax.axis_index("s")

    @pl.when(jnp.logical_and(c == 0, s == 0))
    def _():
        def zero_body(i, _):
            st_v[pl.ds(i * 16, 16)] = jnp.zeros((16,), jnp.int32)
            sg_v[pl.ds(i * 16, 16)] = jnp.zeros((16,), jnp.float32)
            return 0
        lax.fori_loop(0, S // 16, zero_body, 0)
        for d_hbm, g_hbm in ((d1, g1), (d2, g2)):
            pltpu.sync_copy(d_hbm, dst_v)
            pltpu.sync_copy(g_hbm, g_v)

            def sc_body(i, _):
                idx = dst_v[pl.ds(i * 16, 16)]
                tok = lax.iota(jnp.int32, 16) + i * 16
                gv = g_v[pl.ds(i * 16, 16)]
                plsc.store_scatter(st_v, [idx], tok)
                plsc.store_scatter(sg_v, [idx], gv)
                return 0
            lax.fori_loop(0, N // 16, sc_body, 0)
        pltpu.sync_copy(st_v, st_out)
        pltpu.sync_copy(sg_v, sg_out)


def _k2a(*args):
    return pl.kernel(
        _k2a_body,
        (
            jax.ShapeDtypeStruct((S,), jnp.int32),
            jax.ShapeDtypeStruct((S,), jnp.float32),
        ),
        mesh=_sc_mesh(),
        scratch_types=[
            pltpu.VMEM((N,), jnp.int32),
            pltpu.VMEM((N,), jnp.float32),
            pltpu.VMEM((S,), jnp.int32),
            pltpu.VMEM((S,), jnp.float32),
        ],
        compiler_params=pltpu.CompilerParams(needs_layout_passes=False),
    )(*args)


_GCH = 64


def _k2b_body(st, x_hbm, xs_out, idx_v, rows_v, sem):
    c = lax.axis_index("c")
    s = lax.axis_index("s")
    wid = s * 2 + c
    base = wid * (S // NSC)
    for ch in range(S // NSC // _GCH):
        b = pl.multiple_of(base + ch * _GCH, _GCH)
        pltpu.sync_copy(st.at[pl.ds(b, _GCH)], idx_v)
        pltpu.async_copy(x_hbm.at[idx_v], rows_v, sem).wait()
        pltpu.sync_copy(rows_v, xs_out.at[pl.ds(b, _GCH)])


def _k2b(*args):
    return pl.kernel(
        _k2b_body,
        jax.ShapeDtypeStruct((S, D), jnp.float32),
        mesh=_sc_mesh(),
        scratch_types=[
            pltpu.VMEM((_GCH,), jnp.int32),
            pltpu.VMEM((_GCH, D), jnp.float32),
            pltpu.SemaphoreType.DMA,
        ],
    )(*args)


def _k3_body(te_ref, x_ref, w1_ref, b1_ref, w2_ref, b2_ref, g_ref, y_ref):
    j = pl.program_id(0)

    @pl.when(te_ref[j] < E)
    def _():
        h = jnp.dot(x_ref[...], w1_ref[0],
                    preferred_element_type=jnp.float32) + b1_ref[0]
        h = jax.nn.gelu(h)
        y = jnp.dot(h, w2_ref[0],
                    preferred_element_type=jnp.float32) + b2_ref[0]
        y_ref[...] = g_ref[...] * jnp.exp(y)


def _k3(texp, xs, fc1_w, fc1_b, fc2_w, fc2_b, sg):
    def emap(j, t):
        return (jnp.minimum(t[j], E - 1), 0, 0)

    grid_spec = pltpu.PrefetchScalarGridSpec(
        num_scalar_prefetch=1,
        grid=(MAXT,),
        in_specs=[
            pl.BlockSpec((TILE, D), lambda j, t: (j, 0)),
            pl.BlockSpec((1, D, H), emap),
            pl.BlockSpec((1, 1, H), emap),
            pl.BlockSpec((1, H, D), emap),
            pl.BlockSpec((1, 1, D), emap),
            pl.BlockSpec((TILE, 1), lambda j, t: (j, 0)),
        ],
        out_specs=pl.BlockSpec((TILE, D), lambda j, t: (j, 0)),
    )
    return pl.pallas_call(
        _k3_body,
        grid_spec=grid_spec,
        out_shape=jax.ShapeDtypeStruct((S, D), jnp.float32),
    )(texp, xs, fc1_w, fc1_b.reshape(E, 1, H), fc2_w,
      fc2_b.reshape(E, 1, D), sg)


_CCH = 32


def _k4_body(d1, d2, ysc_hbm, out_hbm, ia_v, ib_v, ra_v, rb_v, sem):
    c = lax.axis_index("c")
    s = lax.axis_index("s")
    wid = s * 2 + c
    base = wid * (N // NSC)
    for ch in range(N // NSC // _CCH):
        b = pl.multiple_of(base + ch * _CCH, _CCH)
        pltpu.sync_copy(d1.at[pl.ds(b, _CCH)], ia_v)
        pltpu.sync_copy(d2.at[pl.ds(b, _CCH)], ib_v)
        cp_a = pltpu.async_copy(ysc_hbm.at[ia_v], ra_v, sem)
        cp_b = pltpu.async_copy(ysc_hbm.at[ib_v], rb_v, sem)
        cp_a.wait()
        cp_b.wait()

        def row_body(r, _):
            def lane_body(l, _):
                a = ra_v[r, pl.ds(l * 16, 16)]
                bb = rb_v[r, pl.ds(l * 16, 16)]
                ra_v[r, pl.ds(l * 16, 16)] = a + bb
                return 0
            lax.fori_loop(0, D // 16, lane_body, 0)
            return 0
        lax.fori_loop(0, _CCH, row_body, 0)
        pltpu.sync_copy(ra_v, out_hbm.at[pl.ds(b, _CCH)])


def _k4(*args):
    return pl.kernel(
        _k4_body,
        jax.ShapeDtypeStruct((N, D), jnp.float32),
        mesh=_sc_mesh(),
        scratch_types=[
            pltpu.VMEM((_CCH,), jnp.int32),
            pltpu.VMEM((_CCH,), jnp.int32),
            pltpu.VMEM((_CCH, D), jnp.float32),
            pltpu.VMEM((_CCH, D), jnp.float32),
            pltpu.SemaphoreType.DMA,
        ],
    )(*args)


_EPS = 2.220446049250313e-16


def _k5_body(c_ref, o_ref):
    cv = c_ref[...]
    o_ref[...] = jnp.log(jnp.where(cv == 0.0, _EPS, cv))


def _k5(comb):
    return pl.pallas_call(
        _k5_body,
        grid=(N // TILE,),
        in_specs=[pl.BlockSpec((TILE, D), lambda i: (i, 0))],
        out_specs=pl.BlockSpec((TILE, D), lambda i: (i, 0)),
        out_shape=jax.ShapeDtypeStruct((N, D), jnp.float32),
    )(comb)


def kernel(x, gate_inp, w_gate, fc1_w, fc1_b, fc2_w, fc2_b):
    d1, d2, g1, g2, texp = _k1(gate_inp, w_gate)
    d1f = d1.reshape(N)
    d2f = d2.reshape(N)
    st, sg = _k2a(d1f, d2f, g1.reshape(N), g2.reshape(N))
    xs = _k2b(st, x)
    ysc = _k3(texp.reshape(MAXT), xs, fc1_w, fc1_b, fc2_w, fc2_b,
              sg.reshape(S, 1))
    comb = _k4(d1f, d2f, ysc)
    return _k5(comb)


# speedup vs baseline: 1.0498x; 1.0498x over previous
import functools

import jax
import jax.numpy as jnp
from jax import lax
from jax.experimental import pallas as pl
from jax.experimental.pallas import tpu as pltpu
from jax.experimental.pallas import tpu_sc as plsc

N = 2048
D = 768
H = 3072
E = 8
TILE = 256
MAXT = 24
S = MAXT * TILE
NSC = 32


def _k1_body(gi_ref, wg_ref, d1_ref, d2_ref, g1_ref, g2_ref, te_ref):
    logits = jnp.dot(gi_ref[...], wg_ref[...],
                     preferred_element_type=jnp.float32)
    iota_e = lax.broadcasted_iota(jnp.int32, (N, E), 1)
    m1 = jnp.max(logits, axis=1, keepdims=True)
    i1 = jnp.min(jnp.where(logits == m1, iota_e, E), axis=1, keepdims=True)
    mask1 = iota_e == i1
    logits2 = jnp.where(mask1, -jnp.inf, logits)
    m2 = jnp.max(logits2, axis=1, keepdims=True)
    i2 = jnp.min(jnp.where(logits2 == m2, iota_e, E), axis=1, keepdims=True)
    mask2 = iota_e == i2
    t = jnp.exp(m2 - m1)
    g1 = 1.0 / (1.0 + t)
    g2 = 1.0 - g1
    oh1 = mask1.astype(jnp.float32)
    oh2 = mask2.astype(jnp.float32)

    def incl_cumsum_rows(a):
        sh = 1
        while sh < N:
            a = a + jnp.concatenate(
                [jnp.zeros((sh, E), jnp.float32), a[: N - sh]], axis=0)
            sh *= 2
        return a

    c1 = incl_cumsum_rows(oh1)
    c2 = incl_cumsum_rows(oh2)
    csum1 = c1 - oh1
    csum2 = c2 - oh2
    cnt1 = c1[N - 1:N, :]
    cnt = cnt1 + c2[N - 1:N, :]
    cnt_i = cnt.astype(jnp.int32)
    pc = (((cnt_i + (TILE - 1)) >> 8) << 8).astype(jnp.float32)
    pi = pc
    sh = 1
    while sh < E:
        pi = pi + jnp.concatenate(
            [jnp.zeros((1, sh), jnp.float32), pi[:, : E - sh]], axis=1)
        sh *= 2
    offpad = pi - pc
    ends = pi
    dest1 = jnp.sum(oh1 * (offpad + csum1), axis=1, keepdims=True)
    dest2 = jnp.sum(oh2 * (offpad + cnt1 + csum2), axis=1, keepdims=True)
    d1_ref[...] = dest1.astype(jnp.int32)
    d2_ref[...] = dest2.astype(jnp.int32)
    g1_ref[...] = g1
    g2_ref[...] = g2
    tstart = (lax.broadcasted_iota(jnp.int32, (MAXT, 1), 0) * TILE
              ).astype(jnp.float32)
    te_ref[...] = jnp.sum((tstart >= ends).astype(jnp.int32), axis=1,
                          keepdims=True)


def _k1(gate_inp, w_gate):
    return pl.pallas_call(
        _k1_body,
        out_shape=(
            jax.ShapeDtypeStruct((N, 1), jnp.int32),
            jax.ShapeDtypeStruct((N, 1), jnp.int32),
            jax.ShapeDtypeStruct((N, 1), jnp.float32),
            jax.ShapeDtypeStruct((N, 1), jnp.float32),
            jax.ShapeDtypeStruct((MAXT, 1), jnp.int32),
        ),
    )(gate_inp, w_gate)


@functools.cache
def _sc_mesh():
    return plsc.VectorSubcoreMesh(core_axis_name="c", subcore_axis_name="s")


_GCH = 64
_RPT = S // NSC


def _k2_body(d1, d2, g1, g2, x_hbm, sg_out, xs_out,
             dst_v, g_v, st_v, sg_v, st_sh, idx_v, rows0, rows1, gsem, wsem):
    c = lax.axis_index("c")
    s = lax.axis_index("s")
    wid = s * 2 + c

    @pl.when(s == 0)
    def _():
        def zero_body(i, _):
            st_v[pl.ds(i * 16, 16)] = jnp.zeros((16,), jnp.int32)
            return 0
        lax.fori_loop(0, S // 16, zero_body, 0)
        for d_hbm in (d1, d2):
            pltpu.sync_copy(d_hbm, dst_v)

            def sc_body(i, _):
                idx = dst_v[pl.ds(i * 16, 16)]
                tok = lax.iota(jnp.int32, 16) + i * 16
                plsc.store_scatter(st_v, [idx], tok)
                return 0
            lax.fori_loop(0, N // 16, sc_body, 0)
        pltpu.sync_copy(st_v, st_sh)

    @pl.when(s == 1)
    def _():
        def zero_body(i, _):
            sg_v[pl.ds(i * 16, 16)] = jnp.zeros((16,), jnp.float32)
            return 0
        lax.fori_loop(0, S // 16, zero_body, 0)
        for d_hbm, g_hbm in ((d1, g1), (d2, g2)):
            pltpu.sync_copy(d_hbm, dst_v)
            pltpu.sync_copy(g_hbm, g_v)

            def sc_body(i, _):
                idx = dst_v[pl.ds(i * 16, 16)]
                gv = g_v[pl.ds(i * 16, 16)]
                plsc.store_scatter(sg_v, [idx], gv)
                return 0
            lax.fori_loop(0, N // 16, sc_body, 0)
        half = pl.multiple_of(c * (S // 2), 8)
        pltpu.sync_copy(sg_v.at[pl.ds(half, S // 2)],
                        sg_out.at[pl.ds(half, S // 2)])

    plsc.subcore_barrier()
    base = pl.multiple_of(wid * _RPT, 8)
    pltpu.sync_copy(st_sh.at[pl.ds(base, _RPT)], idx_v)
    g0 = pltpu.async_copy(x_hbm.at[idx_v.at[pl.ds(0, _GCH)]], rows0, gsem)
    g1 = pltpu.async_copy(x_hbm.at[idx_v.at[pl.ds(_GCH, _GCH)]], rows1, gsem)
    g0.wait()
    w0 = pltpu.async_copy(rows0, xs_out.at[pl.ds(base, _GCH)], wsem)
    g1.wait()
    w0.wait()
    g2 = pltpu.async_copy(x_hbm.at[idx_v.at[pl.ds(2 * _GCH, _GCH)]], rows0,
                          gsem)
    w1 = pltpu.async_copy(rows1, xs_out.at[pl.ds(base + _GCH, _GCH)], wsem)
    g2.wait()
    w2 = pltpu.async_copy(rows0, xs_out.at[pl.ds(base + 2 * _GCH, _GCH)], wsem)
    w1.wait()
    w2.wait()


def _k2(*args):
    return pl.kernel(
        _k2_body,
        (
            jax.ShapeDtypeStruct((S,), jnp.float32),
            jax.ShapeDtypeStruct((S, D), jnp.float32),
        ),
        mesh=_sc_mesh(),
        scratch_types=[
            pltpu.VMEM((N,), jnp.int32),
            pltpu.VMEM((N,), jnp.float32),
            pltpu.VMEM((S,), jnp.int32),
            pltpu.VMEM((S,), jnp.float32),
            pltpu.VMEM_SHARED((S,), jnp.int32),
            pltpu.VMEM((_RPT,), jnp.int32),
            pltpu.VMEM((_GCH, D), jnp.float32),
            pltpu.VMEM((_GCH, D), jnp.float32),
            pltpu.SemaphoreType.DMA,
            pltpu.SemaphoreType.DMA,
        ],
        compiler_params=pltpu.CompilerParams(needs_layout_passes=False),
    )(*args)


def _k3_body(te_ref, x_ref, w1_ref, b1_ref, w2_ref, b2_ref, g_ref, y_ref):
    j = pl.program_id(0)

    @pl.when(te_ref[j] < E)
    def _():
        h = jnp.dot(x_ref[...], w1_ref[0],
                    preferred_element_type=jnp.float32) + b1_ref[0]
        h = jax.nn.gelu(h)
        y = jnp.dot(h, w2_ref[0],
                    preferred_element_type=jnp.float32) + b2_ref[0]
        y_ref[...] = g_ref[...] * jnp.exp(y)


def _k3(texp, xs, fc1_w, fc1_b, fc2_w, fc2_b, sg):
    def emap(j, t):
        return (jnp.minimum(t[j], E - 1), 0, 0)

    grid_spec = pltpu.PrefetchScalarGridSpec(
        num_scalar_prefetch=1,
        grid=(MAXT,),
        in_specs=[
            pl.BlockSpec((TILE, D), lambda j, t: (j, 0)),
            pl.BlockSpec((1, D, H), emap),
            pl.BlockSpec((1, 1, H), emap),
            pl.BlockSpec((1, H, D), emap),
            pl.BlockSpec((1, 1, D), emap),
            pl.BlockSpec((TILE, 1), lambda j, t: (j, 0)),
        ],
        out_specs=pl.BlockSpec((TILE, D), lambda j, t: (j, 0)),
    )
    return pl.pallas_call(
        _k3_body,
        grid_spec=grid_spec,
        out_shape=jax.ShapeDtypeStruct((S, D), jnp.float32),
    )(texp, xs, fc1_w, fc1_b.reshape(E, 1, H), fc2_w,
      fc2_b.reshape(E, 1, D), sg)


_CCH = 32


def _k4_body(d1, d2, ysc_hbm, out_hbm, ia_v, ib_v,
             a0, b0, a1, b1, gsem, wsem):
    c = lax.axis_index("c")
    s = lax.axis_index("s")
    wid = s * 2 + c
    base = pl.multiple_of(wid * (N // NSC), 8)
    pltpu.sync_copy(d1.at[pl.ds(base, 2 * _CCH)], ia_v)
    pltpu.sync_copy(d2.at[pl.ds(base, 2 * _CCH)], ib_v)
    ca0 = pltpu.async_copy(ysc_hbm.at[ia_v.at[pl.ds(0, _CCH)]], a0, gsem)
    cb0 = pltpu.async_copy(ysc_hbm.at[ib_v.at[pl.ds(0, _CCH)]], b0, gsem)
    ca1 = pltpu.async_copy(ysc_hbm.at[ia_v.at[pl.ds(_CCH, _CCH)]], a1, gsem)
    cb1 = pltpu.async_copy(ysc_hbm.at[ib_v.at[pl.ds(_CCH, _CCH)]], b1, gsem)

    def add_rows(av, bv):
        def row_body(r, _):
            for l in range(D // 16):
                av[r, pl.ds(l * 16, 16)] = (av[r, pl.ds(l * 16, 16)]
                                            + bv[r, pl.ds(l * 16, 16)])
            return 0
        lax.fori_loop(0, _CCH, row_body, 0)

    ca0.wait()
    cb0.wait()
    add_rows(a0, b0)
    w0 = pltpu.async_copy(a0, out_hbm.at[pl.ds(base, _CCH)], wsem)
    ca1.wait()
    cb1.wait()
    add_rows(a1, b1)
    w1 = pltpu.async_copy(a1, out_hbm.at[pl.ds(base + _CCH, _CCH)], wsem)
    w0.wait()
    w1.wait()


def _k4(*args):
    return pl.kernel(
        _k4_body,
        jax.ShapeDtypeStruct((N, D), jnp.float32),
        mesh=_sc_mesh(),
        scratch_types=[
            pltpu.VMEM((2 * _CCH,), jnp.int32),
            pltpu.VMEM((2 * _CCH,), jnp.int32),
            pltpu.VMEM((_CCH, D), jnp.float32),
            pltpu.VMEM((_CCH, D), jnp.float32),
            pltpu.VMEM((_CCH, D), jnp.float32),
            pltpu.VMEM((_CCH, D), jnp.float32),
            pltpu.SemaphoreType.DMA,
            pltpu.SemaphoreType.DMA,
        ],
        compiler_params=pltpu.CompilerParams(needs_layout_passes=False),
    )(*args)


_EPS = 2.220446049250313e-16


def _k5_body(c_ref, o_ref):
    cv = c_ref[...]
    o_ref[...] = jnp.log(jnp.where(cv == 0.0, _EPS, cv))


def _k5(comb):
    return pl.pallas_call(
        _k5_body,
        grid=(N // TILE,),
        in_specs=[pl.BlockSpec((TILE, D), lambda i: (i, 0))],
        out_specs=pl.BlockSpec((TILE, D), lambda i: (i, 0)),
        out_shape=jax.ShapeDtypeStruct((N, D), jnp.float32),
    )(comb)


def kernel(x, gate_inp, w_gate, fc1_w, fc1_b, fc2_w, fc2_b):
    d1, d2, g1, g2, texp = _k1(gate_inp, w_gate)
    d1f = d1.reshape(N)
    d2f = d2.reshape(N)
    sg, xs = _k2(d1f, d2f, g1.reshape(N), g2.reshape(N), x)
    ysc = _k3(texp.reshape(MAXT), xs, fc1_w, fc1_b, fc2_w, fc2_b,
              sg.reshape(S, 1))
    comb = _k4(d1f, d2f, ysc)
    return _k5(comb)


# speedup vs baseline: 1.1960x; 1.1393x over previous
import functools

import jax
import jax.numpy as jnp
from jax import lax
from jax.experimental import pallas as pl
from jax.experimental.pallas import tpu as pltpu
from jax.experimental.pallas import tpu_sc as plsc

N = 2048
D = 768
H = 3072
E = 8
TILE = 256
MAXT = 24
S = MAXT * TILE
NSC = 32


def _k1_body(gi_ref, wg_ref, d1_ref, d2_ref, g1_ref, g2_ref, te_ref):
    logits = jnp.dot(gi_ref[...], wg_ref[...],
                     preferred_element_type=jnp.float32)
    iota_e = lax.broadcasted_iota(jnp.int32, (N, E), 1)
    m1 = jnp.max(logits, axis=1, keepdims=True)
    i1 = jnp.min(jnp.where(logits == m1, iota_e, E), axis=1, keepdims=True)
    mask1 = iota_e == i1
    logits2 = jnp.where(mask1, -jnp.inf, logits)
    m2 = jnp.max(logits2, axis=1, keepdims=True)
    i2 = jnp.min(jnp.where(logits2 == m2, iota_e, E), axis=1, keepdims=True)
    mask2 = iota_e == i2
    t = jnp.exp(m2 - m1)
    g1 = 1.0 / (1.0 + t)
    g2 = 1.0 - g1
    oh1 = mask1.astype(jnp.float32)
    oh2 = mask2.astype(jnp.float32)

    def incl_cumsum_rows(a):
        sh = 1
        while sh < N:
            a = a + jnp.concatenate(
                [jnp.zeros((sh, E), jnp.float32), a[: N - sh]], axis=0)
            sh *= 2
        return a

    c1 = incl_cumsum_rows(oh1)
    c2 = incl_cumsum_rows(oh2)
    csum1 = c1 - oh1
    csum2 = c2 - oh2
    cnt1 = c1[N - 1:N, :]
    cnt = cnt1 + c2[N - 1:N, :]
    cnt_i = cnt.astype(jnp.int32)
    pc = (((cnt_i + (TILE - 1)) >> 8) << 8).astype(jnp.float32)
    pi = pc
    sh = 1
    while sh < E:
        pi = pi + jnp.concatenate(
            [jnp.zeros((1, sh), jnp.float32), pi[:, : E - sh]], axis=1)
        sh *= 2
    offpad = pi - pc
    ends = pi
    dest1 = jnp.sum(oh1 * (offpad + csum1), axis=1, keepdims=True)
    dest2 = jnp.sum(oh2 * (offpad + cnt1 + csum2), axis=1, keepdims=True)
    d1_ref[...] = dest1.astype(jnp.int32)
    d2_ref[...] = dest2.astype(jnp.int32)
    g1_ref[...] = g1
    g2_ref[...] = g2
    tstart = (lax.broadcasted_iota(jnp.int32, (MAXT, 1), 0) * TILE
              ).astype(jnp.float32)
    te_ref[...] = jnp.sum((tstart >= ends).astype(jnp.int32), axis=1,
                          keepdims=True)


def _k1(gate_inp, w_gate):
    return pl.pallas_call(
        _k1_body,
        out_shape=(
            jax.ShapeDtypeStruct((N, 1), jnp.int32),
            jax.ShapeDtypeStruct((N, 1), jnp.int32),
            jax.ShapeDtypeStruct((N, 1), jnp.float32),
            jax.ShapeDtypeStruct((N, 1), jnp.float32),
            jax.ShapeDtypeStruct((MAXT, 1), jnp.int32),
        ),
    )(gate_inp, w_gate)


@functools.cache
def _sc_mesh():
    return plsc.VectorSubcoreMesh(core_axis_name="c", subcore_axis_name="s")


def _k2_body(d1, d2, g1, g2, st_out, sg_out, dst_v, g_v, st_v, sg_v):
    c = lax.axis_index("c")
    s = lax.axis_index("s")
    half = pl.multiple_of(c * (S // 2), 8)

    @pl.when(s == 0)
    def _():
        def zero_body(i, _):
            st_v[pl.ds(i * 16, 16)] = jnp.zeros((16,), jnp.int32)
            return 0
        lax.fori_loop(0, S // 16, zero_body, 0)
        for d_hbm in (d1, d2):
            pltpu.sync_copy(d_hbm, dst_v)

            def sc_body(i, _):
                idx = dst_v[pl.ds(i * 16, 16)]
                tok = lax.iota(jnp.int32, 16) + i * 16
                plsc.store_scatter(st_v, [idx], tok)
                return 0
            lax.fori_loop(0, N // 16, sc_body, 0)
        pltpu.sync_copy(st_v.at[pl.ds(half, S // 2)],
                        st_out.at[pl.ds(half, S // 2)])

    @pl.when(s == 1)
    def _():
        def zero_body(i, _):
            sg_v[pl.ds(i * 16, 16)] = jnp.zeros((16,), jnp.float32)
            return 0
        lax.fori_loop(0, S // 16, zero_body, 0)
        for d_hbm, g_hbm in ((d1, g1), (d2, g2)):
            pltpu.sync_copy(d_hbm, dst_v)
            pltpu.sync_copy(g_hbm, g_v)

            def sc_body(i, _):
                idx = dst_v[pl.ds(i * 16, 16)]
                gv = g_v[pl.ds(i * 16, 16)]
                plsc.store_scatter(sg_v, [idx], gv)
                return 0
            lax.fori_loop(0, N // 16, sc_body, 0)
        pltpu.sync_copy(sg_v.at[pl.ds(half, S // 2)],
                        sg_out.at[pl.ds(half, S // 2)])


def _k2(*args):
    return pl.kernel(
        _k2_body,
        (
            jax.ShapeDtypeStruct((S,), jnp.int32),
            jax.ShapeDtypeStruct((S,), jnp.float32),
        ),
        mesh=_sc_mesh(),
        scratch_types=[
            pltpu.VMEM((N,), jnp.int32),
            pltpu.VMEM((N,), jnp.float32),
            pltpu.VMEM((S,), jnp.int32),
            pltpu.VMEM((S,), jnp.float32),
        ],
        compiler_params=pltpu.CompilerParams(needs_layout_passes=False),
    )(*args)


def _k3_body(te_ref, tok_ref, x_ref, w1_ref, b1_ref, w2_ref, b2_ref, g_ref,
             y_ref):
    j = pl.program_id(0)

    @pl.when(te_ref[j] < E)
    def _():
        onehot = (lax.broadcasted_iota(jnp.int32, (TILE, N), 1)
                  == tok_ref[...]).astype(jnp.bfloat16)
        xg = jnp.dot(onehot, x_ref[...],
                     preferred_element_type=jnp.float32)
        h = jnp.dot(xg.astype(jnp.bfloat16), w1_ref[0],
                    preferred_element_type=jnp.float32) + b1_ref[0]
        h = jax.nn.gelu(h)
        y = jnp.dot(h.astype(jnp.bfloat16), w2_ref[0],
                    preferred_element_type=jnp.float32) + b2_ref[0]
        y_ref[...] = g_ref[...] * jnp.exp(y)


def _k3(texp, st, x, fc1_w, fc1_b, fc2_w, fc2_b, sg):
    def emap(j, t):
        return (jnp.minimum(t[j], E - 1), 0, 0)

    grid_spec = pltpu.PrefetchScalarGridSpec(
        num_scalar_prefetch=1,
        grid=(MAXT,),
        in_specs=[
            pl.BlockSpec((TILE, 1), lambda j, t: (j, 0)),
            pl.BlockSpec((N, D), lambda j, t: (0, 0)),
            pl.BlockSpec((1, D, H), emap),
            pl.BlockSpec((1, 1, H), emap),
            pl.BlockSpec((1, H, D), emap),
            pl.BlockSpec((1, 1, D), emap),
            pl.BlockSpec((TILE, 1), lambda j, t: (j, 0)),
        ],
        out_specs=pl.BlockSpec((TILE, D), lambda j, t: (j, 0)),
    )
    return pl.pallas_call(
        _k3_body,
        grid_spec=grid_spec,
        out_shape=jax.ShapeDtypeStruct((S, D), jnp.float32),
    )(texp, st, x.astype(jnp.bfloat16),
      fc1_w.astype(jnp.bfloat16), fc1_b.reshape(E, 1, H),
      fc2_w.astype(jnp.bfloat16), fc2_b.reshape(E, 1, D), sg)


_CCH = 32


def _k4_body(d1, d2, ysc_hbm, out_hbm, ia_v, ib_v,
             a0, b0, a1, b1, gsem, wsem):
    c = lax.axis_index("c")
    s = lax.axis_index("s")
    wid = s * 2 + c
    base = pl.multiple_of(wid * (N // NSC), 8)
    pltpu.sync_copy(d1.at[pl.ds(base, 2 * _CCH)], ia_v)
    pltpu.sync_copy(d2.at[pl.ds(base, 2 * _CCH)], ib_v)
    ca0 = pltpu.async_copy(ysc_hbm.at[ia_v.at[pl.ds(0, _CCH)]], a0, gsem)
    cb0 = pltpu.async_copy(ysc_hbm.at[ib_v.at[pl.ds(0, _CCH)]], b0, gsem)
    ca1 = pltpu.async_copy(ysc_hbm.at[ia_v.at[pl.ds(_CCH, _CCH)]], a1, gsem)
    cb1 = pltpu.async_copy(ysc_hbm.at[ib_v.at[pl.ds(_CCH, _CCH)]], b1, gsem)

    def add_rows(av, bv):
        def row_body(r, _):
            for l in range(D // 16):
                av[r, pl.ds(l * 16, 16)] = (av[r, pl.ds(l * 16, 16)]
                                            + bv[r, pl.ds(l * 16, 16)])
            return 0
        lax.fori_loop(0, _CCH, row_body, 0)

    ca0.wait()
    cb0.wait()
    add_rows(a0, b0)
    w0 = pltpu.async_copy(a0, out_hbm.at[pl.ds(base, _CCH)], wsem)
    ca1.wait()
    cb1.wait()
    add_rows(a1, b1)
    w1 = pltpu.async_copy(a1, out_hbm.at[pl.ds(base + _CCH, _CCH)], wsem)
    w0.wait()
    w1.wait()


def _k4(*args):
    return pl.kernel(
        _k4_body,
        jax.ShapeDtypeStruct((N, D), jnp.float32),
        mesh=_sc_mesh(),
        scratch_types=[
            pltpu.VMEM((2 * _CCH,), jnp.int32),
            pltpu.VMEM((2 * _CCH,), jnp.int32),
            pltpu.VMEM((_CCH, D), jnp.float32),
            pltpu.VMEM((_CCH, D), jnp.float32),
            pltpu.VMEM((_CCH, D), jnp.float32),
            pltpu.VMEM((_CCH, D), jnp.float32),
            pltpu.SemaphoreType.DMA,
            pltpu.SemaphoreType.DMA,
        ],
        compiler_params=pltpu.CompilerParams(needs_layout_passes=False),
    )(*args)


_EPS = 2.220446049250313e-16


def _k5_body(c_ref, o_ref):
    cv = c_ref[...]
    o_ref[...] = jnp.log(jnp.where(cv == 0.0, _EPS, cv))


def _k5(comb):
    return pl.pallas_call(
        _k5_body,
        grid=(N // TILE,),
        in_specs=[pl.BlockSpec((TILE, D), lambda i: (i, 0))],
        out_specs=pl.BlockSpec((TILE, D), lambda i: (i, 0)),
        out_shape=jax.ShapeDtypeStruct((N, D), jnp.float32),
    )(comb)


def kernel(x, gate_inp, w_gate, fc1_w, fc1_b, fc2_w, fc2_b):
    d1, d2, g1, g2, texp = _k1(gate_inp, w_gate)
    d1f = d1.reshape(N)
    d2f = d2.reshape(N)
    st, sg = _k2(d1f, d2f, g1.reshape(N), g2.reshape(N))
    ysc = _k3(texp.reshape(MAXT), st.reshape(S, 1), x, fc1_w, fc1_b,
              fc2_w, fc2_b, sg.reshape(S, 1))
    comb = _k4(d1f, d2f, ysc)
    return _k5(comb)
